# Initial kernel scaffold; baseline (speedup 1.0000x reference)
import jax
import jax.numpy as jnp
from jax.experimental import pallas as pl


def kernel(x, W_hour, W_day, W_weekday, W_month):
    raise NotImplementedError("write your pallas kernel here")


# speedup vs baseline: 7.1787x; 7.1787x over previous
import functools

import jax
import jax.numpy as jnp
from jax import lax
from jax.experimental import pallas as pl
from jax.experimental.pallas import tpu as pltpu
from jax.experimental.pallas import tpu_sc as plsc

D = 128
NF = 4
V = 32
LANES = 16
NC, NS = 2, 16
NW = NC * NS
T = 256


def _sc_embed(n_tokens):
    tpw = n_tokens // NW
    g_count = tpw // T

    mesh = plsc.VectorSubcoreMesh(
        core_axis_name="c", subcore_axis_name="s",
        num_cores=NC, num_subcores=NS)

    @functools.partial(
        pl.kernel,
        out_type=jax.ShapeDtypeStruct((n_tokens, D), jnp.float32),
        mesh=mesh,
        scratch_types=[
            pltpu.VMEM((NF, V, D), jnp.float32),
            pltpu.VMEM((2, T * NF), jnp.int32),
            pltpu.VMEM((2, T, D), jnp.float32),
            pltpu.SemaphoreType.DMA,
            pltpu.SemaphoreType.DMA,
            pltpu.SemaphoreType.DMA,
            pltpu.SemaphoreType.DMA,
        ],
    )
    def body(x_hbm, w_hbm, o_hbm, w_v, idx_v, out_v, si0, si1, so0, so1):
        sin = (si0, si1)
        sout = (so0, so1)
        wid = lax.axis_index("s") * NC + lax.axis_index("c")
        base = wid * tpw

        pltpu.async_copy(
            x_hbm.at[pl.ds(base * NF, T * NF)], idx_v.at[0], sin[0])
        pltpu.async_copy(
            x_hbm.at[pl.ds((base + T) * NF, T * NF)], idx_v.at[1], sin[1])
        pltpu.sync_copy(w_hbm, w_v)

        @pl.loop(0, g_count, step=2)
        def _(g0):
            for p in range(2):
                g = g0 + p
                start = base + g * T
                pltpu.make_async_copy(
                    x_hbm.at[pl.ds(start * NF, T * NF)],
                    idx_v.at[p], sin[p]).wait()

                @pl.when(g >= 2)
                def _():
                    pltpu.make_async_copy(
                        out_v.at[p],
                        o_hbm.at[pl.ds(start - 2 * T, T)],
                        sout[p]).wait()

                @plsc.parallel_loop(0, T // 4, unroll=2)
                def _(q):
                    iv = idx_v[p, pl.ds(q * 16, 16)]
                    for k in range(4):
                        t = q * 4 + k
                        i0 = iv[4 * k]
                        i1 = iv[4 * k + 1]
                        i2 = iv[4 * k + 2]
                        i3 = iv[4 * k + 3]
                        for j in range(D // LANES):
                            sl = pl.ds(j * LANES, LANES)
                            out_v[p, t, sl] = (
                                (w_v[0, i0, sl] + w_v[1, i1, sl])
                                + (w_v[2, i2, sl] + w_v[3, i3, sl]))

                pltpu.async_copy(
                    out_v.at[p], o_hbm.at[pl.ds(start, T)], sout[p])

                @pl.when(g + 2 < g_count)
                def _():
                    pltpu.async_copy(
                        x_hbm.at[pl.ds((start + 2 * T) * NF, T * NF)],
                        idx_v.at[p], sin[p])

        for p in range(2):
            pltpu.make_async_copy(
                out_v.at[p],
                o_hbm.at[pl.ds(base + (g_count - 2 + p) * T, T)],
                sout[p]).wait()

    return body


def kernel(x, W_hour, W_day, W_weekday, W_month):
    b, l, nf = x.shape
    assert nf == NF
    n_tokens = b * l
    assert n_tokens % (NW * T * 2) == 0
    xf = x.reshape(n_tokens * NF)
    w = jnp.stack([W_hour, W_day, W_weekday, W_month])
    out = _sc_embed(n_tokens)(xf, w)
    return out.reshape(b, l, D)
